# baseline (speedup 1.0000x reference)
import functools

import jax
import jax.numpy as jnp
from jax import lax
from jax.experimental import pallas as pl
from jax.experimental.pallas import tpu as pltpu
from jax.experimental.pallas import tpu_sc as plsc

N_NODES = 10000
N_EDGES = 320000
D = 128

ROWS = 2000
GRID = N_NODES // ROWS

NC, NS = 2, 16
EPW = N_EDGES // (NC * NS)
BC = 125
NCH = EPW // BC
N_PAD = 10112
SR = N_PAD // NS


def _ln(t, g, b):
    mu = jnp.mean(t, axis=-1, keepdims=True)
    var = jnp.mean((t - mu) ** 2, axis=-1, keepdims=True)
    return (t - mu) * lax.rsqrt(var + 1e-5) * g + b


def _enc_body(x_ref, w1_ref, b1_ref, w2_ref, b2_ref, w3_ref, b3_ref,
              g_ref, bb_ref, o_ref):
    t = jnp.dot(x_ref[...], w1_ref[...], preferred_element_type=jnp.float32)
    t = jnp.maximum(t + b1_ref[...], 0.0)
    t = jnp.dot(t, w2_ref[...], preferred_element_type=jnp.float32)
    t = jnp.maximum(t + b2_ref[...], 0.0)
    t = jnp.dot(t, w3_ref[...], preferred_element_type=jnp.float32) + b3_ref[...]
    o_ref[...] = _ln(t, g_ref[...], bb_ref[...])


def _node_body(h_ref, agg_ref, w1h_ref, w1a_ref, b1_ref, w2_ref, b2_ref,
               w3_ref, b3_ref, g_ref, bb_ref, o_ref):
    h = h_ref[...]
    agg = jnp.concatenate([agg_ref[0, 0] + agg_ref[1, 0],
                           agg_ref[0, 1] + agg_ref[1, 1]], axis=-1)
    t = jnp.dot(h, w1h_ref[...], preferred_element_type=jnp.float32)
    t = t + jnp.dot(agg, w1a_ref[...], preferred_element_type=jnp.float32)
    t = jnp.maximum(t + b1_ref[...], 0.0)
    t = jnp.dot(t, w2_ref[...], preferred_element_type=jnp.float32)
    t = jnp.maximum(t + b2_ref[...], 0.0)
    t = jnp.dot(t, w3_ref[...], preferred_element_type=jnp.float32) + b3_ref[...]
    o_ref[...] = h + _ln(t, g_ref[...], bb_ref[...])


def _nodedec_body(h_ref, agg_ref, w1h_ref, w1a_ref, b1_ref, w2_ref, b2_ref,
                  w3_ref, b3_ref, g_ref, bb_ref,
                  dw1_ref, db1_ref, dw2_ref, db2_ref, dw3_ref, db3_ref, o_ref):
    h = h_ref[...]
    agg = jnp.concatenate([agg_ref[0, 0] + agg_ref[1, 0],
                           agg_ref[0, 1] + agg_ref[1, 1]], axis=-1)
    t = jnp.dot(h, w1h_ref[...], preferred_element_type=jnp.float32)
    t = t + jnp.dot(agg, w1a_ref[...], preferred_element_type=jnp.float32)
    t = jnp.maximum(t + b1_ref[...], 0.0)
    t = jnp.dot(t, w2_ref[...], preferred_element_type=jnp.float32)
    t = jnp.maximum(t + b2_ref[...], 0.0)
    t = jnp.dot(t, w3_ref[...], preferred_element_type=jnp.float32) + b3_ref[...]
    h = h + _ln(t, g_ref[...], bb_ref[...])
    t = jnp.dot(h, dw1_ref[...], preferred_element_type=jnp.float32)
    t = jnp.maximum(t + db1_ref[...], 0.0)
    t = jnp.dot(t, dw2_ref[...], preferred_element_type=jnp.float32)
    t = jnp.maximum(t + db2_ref[...], 0.0)
    o_ref[...] = jnp.dot(t, dw3_ref[...], preferred_element_type=jnp.float32) + db3_ref[...]


_W = pl.BlockSpec((D, D), lambda i: (0, 0))
_V = pl.BlockSpec((1, D), lambda i: (0, 0))
_R = pl.BlockSpec((ROWS, D), lambda i: (i, 0))
_A = pl.BlockSpec((NC, 2, ROWS, D // 2), lambda i: (0, 0, i, 0))


def _encode(x, p):
    (w1, b1), (w2, b2), (w3, b3) = p["mlp"]
    g, bb = p["ln"]
    return pl.pallas_call(
        _enc_body,
        grid=(GRID,),
        in_specs=[_R, _W, _V, _W, _V, _W, _V, _V, _V],
        out_specs=_R,
        out_shape=jax.ShapeDtypeStruct((N_NODES, D), jnp.float32),
    )(x, w1, b1.reshape(1, D), w2, b2.reshape(1, D), w3, b3.reshape(1, D),
      g.reshape(1, D), bb.reshape(1, D))


def _node_update(h, agg, p):
    (w1, b1), (w2, b2), (w3, b3) = p["mlp"]
    g, bb = p["ln"]
    return pl.pallas_call(
        _node_body,
        grid=(GRID,),
        in_specs=[_R, _A, _W, _W, _V, _W, _V, _W, _V, _V, _V],
        out_specs=_R,
        out_shape=jax.ShapeDtypeStruct((N_NODES, D), jnp.float32),
    )(h, agg, w1[:D], w1[D:], b1.reshape(1, D), w2, b2.reshape(1, D),
      w3, b3.reshape(1, D), g.reshape(1, D), bb.reshape(1, D))


def _node_update_decode(h, agg, p, pd):
    (w1, b1), (w2, b2), (w3, b3) = p["mlp"]
    g, bb = p["ln"]
    (dw1, db1), (dw2, db2), (dw3, db3) = pd
    dw3p = jnp.zeros((D, D), jnp.float32).at[:, : dw3.shape[1]].set(dw3)
    db3p = jnp.zeros((1, D), jnp.float32).at[:, : dw3.shape[1]].set(db3)
    out = pl.pallas_call(
        _nodedec_body,
        grid=(GRID,),
        in_specs=[_R, _A, _W, _W, _V, _W, _V, _W, _V, _V, _V,
                  _W, _V, _W, _V, _W, _V],
        out_specs=_R,
        out_shape=jax.ShapeDtypeStruct((N_NODES, D), jnp.float32),
    )(h, agg, w1[:D], w1[D:], b1.reshape(1, D), w2, b2.reshape(1, D),
      w3, b3.reshape(1, D), g.reshape(1, D), bb.reshape(1, D),
      dw1, db1.reshape(1, D), dw2, db2.reshape(1, D), dw3p, db3p)
    return out[:, : dw3.shape[1]]


_SC_MESH = plsc.VectorSubcoreMesh(core_axis_name="c", subcore_axis_name="s")

NB = 5
NR = NCH // NB
DH = D // 2


@functools.partial(
    pl.kernel,
    mesh=_SC_MESH,
    compiler_params=pltpu.CompilerParams(use_tc_tiling_on_sc=False),
    out_type=jax.ShapeDtypeStruct((NC, 2, N_PAD, DH), jnp.float32),
    scratch_types=[
        pltpu.VMEM((NCH, BC), jnp.int32),
        pltpu.VMEM((NCH, BC), jnp.int32),
        pltpu.VMEM((NB, BC, DH), jnp.float32),
        pltpu.VMEM((SR // 4, DH), jnp.float32),
        pltpu.VMEM_SHARED((N_PAD, DH), jnp.float32),
        pltpu.SemaphoreType.DMA((NB,)),
        pltpu.SemaphoreType.DMA((NB,)),
    ],
)
def _seg_sum(h2_hbm, src_hbm, dst_hbm, z_hbm, out_hbm,
             src_v, dst_v, rows_v, zero_v, agg_sh, gsem, ssem):
    c = lax.axis_index("c")
    s = lax.axis_index("s")
    pltpu.sync_copy(dst_hbm.at[c, s], dst_v)
    pltpu.sync_copy(z_hbm, zero_v)

    for half in range(2):
        pltpu.sync_copy(src_hbm.at[half, c, s], src_v)
        for q in range(4):
            pltpu.sync_copy(zero_v, agg_sh.at[pl.ds(s * SR + q * (SR // 4), SR // 4)])
        plsc.subcore_barrier()

        def gather(j, b):
            pltpu.async_copy(h2_hbm.at[src_v.at[j]], rows_v.at[b], gsem.at[b])

        def gather_wait(j, b):
            pltpu.make_async_copy(h2_hbm.at[src_v.at[j]], rows_v.at[b], gsem.at[b]).wait()

        def scat(j, b):
            pltpu.async_copy(rows_v.at[b], agg_sh.at[dst_v.at[j]], ssem.at[b], add=True)

        def scat_wait(j, b):
            pltpu.make_async_copy(rows_v.at[b], agg_sh.at[dst_v.at[j]], ssem.at[b]).wait()

        for b in range(NB):
            gather(b, b)

        def outer(g, carry):
            base = g * NB
            for b in range(NB):
                gather_wait(base + b, b)
                scat(base + b, b)
            for b in range(NB):
                scat_wait(base + b, b)
                gather(base + NB + b, b)
            return carry

        lax.fori_loop(0, NR - 1, outer, 0)

        base = (NR - 1) * NB
        for b in range(NB):
            gather_wait(base + b, b)
            scat(base + b, b)
        for b in range(NB):
            scat_wait(base + b, b)

        plsc.subcore_barrier()
        pltpu.sync_copy(agg_sh.at[pl.ds(s * SR, SR)],
                        out_hbm.at[c, half, pl.ds(s * SR, SR)])
        plsc.subcore_barrier()


def kernel(x, edge_index, edge_attr, params):
    del edge_attr
    perm = jnp.argsort(edge_index[0])
    src = edge_index[0][perm]
    src2 = jnp.stack([2 * src, 2 * src + 1]).reshape(2, NC, NS, NCH, BC)
    dst4 = edge_index[1][perm].reshape(NC, NS, NCH, BC)
    zeros = jnp.zeros((SR // 4, DH), jnp.float32)
    h = _encode(x, params["enc_node"])
    for p in params["proc"][:-1]:
        agg = _seg_sum(h.reshape(2 * N_NODES, DH), src2, dst4, zeros)
        h = _node_update(h, agg, p["node"])
    agg = _seg_sum(h.reshape(2 * N_NODES, DH), src2, dst4, zeros)
    return _node_update_decode(h, agg, params["proc"][-1]["node"], params["dec"])


# speedup vs baseline: 1.6508x; 1.6508x over previous
import functools

import jax
import jax.numpy as jnp
from jax import lax
from jax.experimental import pallas as pl
from jax.experimental.pallas import tpu as pltpu
from jax.experimental.pallas import tpu_sc as plsc

N_NODES = 10000
N_EDGES = 320000
D = 128

ROWS = 2000
GRID = N_NODES // ROWS

NC, NS = 2, 16
EPW = N_EDGES // (NC * NS)
BC = 125
NCH = EPW // BC
N_PAD = 10112
SR = N_PAD // NS


def _ln(t, g, b):
    mu = jnp.mean(t, axis=-1, keepdims=True)
    var = jnp.mean((t - mu) ** 2, axis=-1, keepdims=True)
    return (t - mu) * lax.rsqrt(var + 1e-5) * g + b


def _enc_body(x_ref, w1_ref, b1_ref, w2_ref, b2_ref, w3_ref, b3_ref,
              g_ref, bb_ref, o_ref):
    t = jnp.dot(x_ref[...], w1_ref[...], preferred_element_type=jnp.float32)
    t = jnp.maximum(t + b1_ref[...], 0.0)
    t = jnp.dot(t, w2_ref[...], preferred_element_type=jnp.float32)
    t = jnp.maximum(t + b2_ref[...], 0.0)
    t = jnp.dot(t, w3_ref[...], preferred_element_type=jnp.float32) + b3_ref[...]
    o_ref[...] = _ln(t, g_ref[...], bb_ref[...])


def _node_body(h_ref, agg_ref, w1_ref, b1_ref, w2_ref, b2_ref,
               w3_ref, b3_ref, g_ref, bb_ref, o_ref):
    h = h_ref[...]
    hin = jnp.concatenate(
        [h, agg_ref[0, 0] + agg_ref[1, 0], agg_ref[0, 1] + agg_ref[1, 1]], axis=-1)
    t = jnp.dot(hin, w1_ref[...], preferred_element_type=jnp.float32)
    t = jnp.maximum(t + b1_ref[...], 0.0)
    t = jnp.dot(t, w2_ref[...], preferred_element_type=jnp.float32)
    t = jnp.maximum(t + b2_ref[...], 0.0)
    t = jnp.dot(t, w3_ref[...], preferred_element_type=jnp.float32) + b3_ref[...]
    o_ref[...] = h + _ln(t, g_ref[...], bb_ref[...])


def _nodedec_body(h_ref, agg_ref, w1h_ref, w1a_ref, b1_ref, w2_ref, b2_ref,
                  w3_ref, b3_ref, g_ref, bb_ref,
                  dw1_ref, db1_ref, dw2_ref, db2_ref, dw3_ref, db3_ref, o_ref):
    h = h_ref[...]
    agg = jnp.concatenate([agg_ref[0, 0] + agg_ref[1, 0],
                           agg_ref[0, 1] + agg_ref[1, 1]], axis=-1)
    t = jnp.dot(h, w1h_ref[...], preferred_element_type=jnp.float32)
    t = t + jnp.dot(agg, w1a_ref[...], preferred_element_type=jnp.float32)
    t = jnp.maximum(t + b1_ref[...], 0.0)
    t = jnp.dot(t, w2_ref[...], preferred_element_type=jnp.float32)
    t = jnp.maximum(t + b2_ref[...], 0.0)
    t = jnp.dot(t, w3_ref[...], preferred_element_type=jnp.float32) + b3_ref[...]
    h = h + _ln(t, g_ref[...], bb_ref[...])
    t = jnp.dot(h, dw1_ref[...], preferred_element_type=jnp.float32)
    t = jnp.maximum(t + db1_ref[...], 0.0)
    t = jnp.dot(t, dw2_ref[...], preferred_element_type=jnp.float32)
    t = jnp.maximum(t + db2_ref[...], 0.0)
    o_ref[...] = jnp.dot(t, dw3_ref[...], preferred_element_type=jnp.float32) + db3_ref[...]


_W = pl.BlockSpec((D, D), lambda i: (0, 0))
_V = pl.BlockSpec((1, D), lambda i: (0, 0))
_R = pl.BlockSpec((ROWS, D), lambda i: (i, 0))
_A = pl.BlockSpec((NC, 2, ROWS, D // 2), lambda i: (0, 0, i, 0))
_W2 = pl.BlockSpec((2 * D, D), lambda i: (0, 0))


def _encode(x, p):
    (w1, b1), (w2, b2), (w3, b3) = p["mlp"]
    g, bb = p["ln"]
    return pl.pallas_call(
        _enc_body,
        grid=(GRID,),
        in_specs=[_R, _W, _V, _W, _V, _W, _V, _V, _V],
        out_specs=_R,
        out_shape=jax.ShapeDtypeStruct((N_NODES, D), jnp.float32),
    )(x, w1, b1.reshape(1, D), w2, b2.reshape(1, D), w3, b3.reshape(1, D),
      g.reshape(1, D), bb.reshape(1, D))


def _node_update(h, agg, p):
    (w1, b1), (w2, b2), (w3, b3) = p["mlp"]
    g, bb = p["ln"]
    return pl.pallas_call(
        _node_body,
        grid=(GRID,),
        in_specs=[_R, _A, _W2, _V, _W, _V, _W, _V, _V, _V],
        out_specs=_R,
        out_shape=jax.ShapeDtypeStruct((N_NODES, D), jnp.float32),
    )(h, agg, w1, b1.reshape(1, D), w2, b2.reshape(1, D),
      w3, b3.reshape(1, D), g.reshape(1, D), bb.reshape(1, D))


def _node_update_decode(h, agg, p, pd):
    (w1, b1), (w2, b2), (w3, b3) = p["mlp"]
    g, bb = p["ln"]
    (dw1, db1), (dw2, db2), (dw3, db3) = pd
    dw3p = jnp.zeros((D, D), jnp.float32).at[:, : dw3.shape[1]].set(dw3)
    db3p = jnp.zeros((1, D), jnp.float32).at[:, : dw3.shape[1]].set(db3)
    out = pl.pallas_call(
        _nodedec_body,
        grid=(GRID,),
        in_specs=[_R, _A, _W, _W, _V, _W, _V, _W, _V, _V, _V,
                  _W, _V, _W, _V, _W, _V],
        out_specs=_R,
        out_shape=jax.ShapeDtypeStruct((N_NODES, D), jnp.float32),
    )(h, agg, w1[:D], w1[D:], b1.reshape(1, D), w2, b2.reshape(1, D),
      w3, b3.reshape(1, D), g.reshape(1, D), bb.reshape(1, D),
      dw1, db1.reshape(1, D), dw2, db2.reshape(1, D), dw3p, db3p)
    return out[:, : dw3.shape[1]]


_SC_MESH = plsc.VectorSubcoreMesh(core_axis_name="c", subcore_axis_name="s")

NB = 5
NR = NCH // NB
DH = D // 2


@functools.partial(
    pl.kernel,
    mesh=_SC_MESH,
    compiler_params=pltpu.CompilerParams(use_tc_tiling_on_sc=False),
    out_type=jax.ShapeDtypeStruct((NC, 2, N_PAD, DH), jnp.float32),
    scratch_types=[
        pltpu.VMEM((NCH, BC), jnp.int32),
        pltpu.VMEM((NCH, BC), jnp.int32),
        pltpu.VMEM((NB, BC, DH), jnp.float32),
        pltpu.VMEM((SR // 4, DH), jnp.float32),
        pltpu.VMEM_SHARED((N_PAD, DH), jnp.float32),
        pltpu.SemaphoreType.DMA((NB,)),
        pltpu.SemaphoreType.DMA((NB,)),
    ],
)
def _seg_sum(h2_hbm, src_hbm, dst_hbm, z_hbm, out_hbm,
             src_v, dst_v, rows_v, zero_v, agg_sh, gsem, ssem):
    c = lax.axis_index("c")
    s = lax.axis_index("s")
    pltpu.sync_copy(dst_hbm.at[c, s], dst_v)
    pltpu.sync_copy(z_hbm, zero_v)

    for half in range(2):
        pltpu.sync_copy(src_hbm.at[half, c, s], src_v)
        for q in range(4):
            pltpu.sync_copy(zero_v, agg_sh.at[pl.ds(s * SR + q * (SR // 4), SR // 4)])
        plsc.subcore_barrier()

        def gather(j, b):
            pltpu.async_copy(h2_hbm.at[src_v.at[j]], rows_v.at[b], gsem.at[b])

        def gather_wait(j, b):
            pltpu.make_async_copy(h2_hbm.at[src_v.at[j]], rows_v.at[b], gsem.at[b]).wait()

        def scat(j, b):
            pltpu.async_copy(rows_v.at[b], agg_sh.at[dst_v.at[j]], ssem.at[b], add=True)

        def scat_wait(j, b):
            pltpu.make_async_copy(rows_v.at[b], agg_sh.at[dst_v.at[j]], ssem.at[b]).wait()

        for b in range(NB):
            gather(b, b)

        def outer(g, carry):
            base = g * NB
            for b in range(NB):
                gather_wait(base + b, b)
                scat(base + b, b)
            for b in range(NB):
                scat_wait(base + b, b)
                gather(base + NB + b, b)
            return carry

        lax.fori_loop(0, NR - 1, outer, 0)

        base = (NR - 1) * NB
        for b in range(NB):
            gather_wait(base + b, b)
            scat(base + b, b)
        for b in range(NB):
            scat_wait(base + b, b)

        plsc.subcore_barrier()
        pltpu.sync_copy(agg_sh.at[pl.ds(s * SR, SR)],
                        out_hbm.at[c, half, pl.ds(s * SR, SR)])
        plsc.subcore_barrier()


def kernel(x, edge_index, edge_attr, params):
    del edge_attr
    src = edge_index[0]
    src2 = jnp.stack([2 * src, 2 * src + 1]).reshape(2, NC, NS, NCH, BC)
    dst4 = edge_index[1].reshape(NC, NS, NCH, BC)
    zeros = jnp.zeros((SR // 4, DH), jnp.float32)
    h = _encode(x, params["enc_node"])
    for p in params["proc"][:-1]:
        agg = _seg_sum(h.reshape(2 * N_NODES, DH), src2, dst4, zeros)
        h = _node_update(h, agg, p["node"])
    agg = _seg_sum(h.reshape(2 * N_NODES, DH), src2, dst4, zeros)
    return _node_update_decode(h, agg, params["proc"][-1]["node"], params["dec"])
